# Initial kernel scaffold; baseline (speedup 1.0000x reference)
import jax
import jax.numpy as jnp
from jax.experimental import pallas as pl


def kernel(x, token_table, pos_table):
    raise NotImplementedError("write your pallas kernel here")


# speedup vs baseline: 1.0962x; 1.0962x over previous
import functools

import jax
import jax.numpy as jnp
from jax import lax
from jax.experimental import pallas as pl
from jax.experimental.pallas import tpu as pltpu
from jax.experimental.pallas import tpu_sc as plsc

NC, NS = 2, 16
NW = NC * NS
B, T, D = 4096, 200, 32
RW = 100
XR = (B * T) // RW
RPW = XR // NW
CH = 8
NCHUNK = RPW // CH
HALF = D // 2


def _sc_embed(x_r, token_table, pos_table):
    mesh = plsc.VectorSubcoreMesh(core_axis_name="c", subcore_axis_name="s")

    @functools.partial(
        pl.kernel,
        out_type=jax.ShapeDtypeStruct((XR, RW, D), jnp.float32),
        mesh=mesh,
        scratch_types=[
            pltpu.VMEM((CH, RW), jnp.int32),
            pltpu.VMEM((CH, RW, D), jnp.float32),
            pltpu.VMEM((T, D), jnp.float32),
            pltpu.SemaphoreType.DMA,
        ],
        compiler_params=pltpu.CompilerParams(use_tc_tiling_on_sc=False),
    )
    def k(x_hbm, tok_hbm, pos_hbm, out_hbm, idx_v, rows_v, pos_v, sem):
        wid = lax.axis_index("s") * NC + lax.axis_index("c")
        base = wid * RPW
        pltpu.sync_copy(pos_hbm, pos_v)

        @pl.loop(0, NCHUNK)
        def _chunk(c):
            r0 = base + c * CH
            pltpu.sync_copy(x_hbm.at[pl.ds(r0, CH)], idx_v)
            for j in range(CH):
                pltpu.async_copy(tok_hbm.at[idx_v.at[j]], rows_v.at[j], sem)
            for j in range(CH):
                pltpu.make_async_copy(tok_hbm.at[idx_v.at[j]], rows_v.at[j], sem).wait()

            @pl.loop(0, RW)
            def _row(r):
                for half in range(2):
                    t = half * RW + r
                    p0 = pos_v[t, 0:HALF]
                    p1 = pos_v[t, HALF:D]
                    for s in range(CH // 2):
                        j = s * 2 + half
                        rows_v[j, r, 0:HALF] += p0
                        rows_v[j, r, HALF:D] += p1

            pltpu.sync_copy(rows_v, out_hbm.at[pl.ds(r0, CH)])

    return k(x_r, token_table, pos_table)


def kernel(x, token_table, pos_table):
    x_r = x.reshape(XR, RW).astype(jnp.int32)
    out = _sc_embed(x_r, token_table, pos_table)
    return out.reshape(B, T, D)


# speedup vs baseline: 1.2454x; 1.1361x over previous
import functools

import jax
import jax.numpy as jnp
from jax import lax
from jax.experimental import pallas as pl
from jax.experimental.pallas import tpu as pltpu
from jax.experimental.pallas import tpu_sc as plsc

NC, NS = 2, 16
NW = NC * NS
B, T, D = 4096, 200, 32
TB = 4
NSTEP = T // TB
DT, DS = D // 8, 8
BT, BS = B // 128, 128


def _sc_embed(x4, token_table, pos_table):
    mesh = plsc.VectorSubcoreMesh(core_axis_name="c", subcore_axis_name="s")

    @functools.partial(
        pl.kernel,
        out_type=jax.ShapeDtypeStruct((T, DT, BT, DS, BS), jnp.float32),
        mesh=mesh,
        scratch_types=[
            pltpu.VMEM((TB, BS), jnp.int32),
            pltpu.VMEM((TB, BS), jnp.int32),
            pltpu.VMEM((TB * BS, D), jnp.float32),
            pltpu.VMEM((TB * BS, D), jnp.float32),
            pltpu.VMEM((TB, DT, DS, BS), jnp.float32),
            pltpu.VMEM((TB, DT, DS, BS), jnp.float32),
            pltpu.VMEM((T, D), jnp.float32),
            pltpu.SemaphoreType.DMA,
            pltpu.SemaphoreType.DMA,
            pltpu.SemaphoreType.DMA,
            pltpu.SemaphoreType.DMA,
            pltpu.SemaphoreType.DMA,
            pltpu.SemaphoreType.DMA,
        ],
        compiler_params=pltpu.CompilerParams(use_tc_tiling_on_sc=False,
                                             needs_layout_passes=False),
    )
    def k(x_hbm, tok_hbm, pos_hbm, out_hbm,
          idx_a, idx_b, rows_a, rows_b, trans_a, trans_b, pos_v,
          si_a, si_b, sg_a, sg_b, so_a, so_b):
        w = lax.axis_index("s") * NC + lax.axis_index("c")
        pltpu.sync_copy(pos_hbm, pos_v)

        iota = lax.iota(jnp.int32, 16)
        row_base = [iota + bsg * 16 for bsg in range(8)]

        def start_idx(i, idx_v, si):
            tt = i // 2
            hs = (i % 2) * TB
            pltpu.async_copy(x_hbm.at[tt, w, pl.ds(hs, TB)], idx_v, si)

        def fire_gathers(idx_v, rows_v, sg):
            for r in range(TB):
                pltpu.async_copy(tok_hbm.at[idx_v.at[r]],
                                 rows_v.at[pl.ds(r * BS, BS)], sg)

        def wait_gathers(idx_v, rows_v, sg):
            for r in range(TB):
                pltpu.make_async_copy(tok_hbm.at[idx_v.at[r]],
                                      rows_v.at[pl.ds(r * BS, BS)], sg).wait()

        def compute(i, rows_v, trans_v):
            @pl.loop(0, TB)
            def _t(tm):
                t = i * TB + tm
                tsplat = jnp.full((16,), t, jnp.int32)
                ridx = [rb + tm * BS for rb in row_base]
                for dt in range(DT):
                    for ds_ in range(DS):
                        d = dt * 8 + ds_
                        dsplat = jnp.full((16,), d, jnp.int32)
                        pvec = plsc.load_gather(pos_v, [tsplat, dsplat])
                        for bsg in range(8):
                            v = plsc.load_gather(rows_v, [ridx[bsg], dsplat])
                            trans_v[tm, dt, ds_, pl.ds(bsg * 16, 16)] = v + pvec

        def start_out(i, trans_v, so):
            pltpu.async_copy(trans_v, out_hbm.at[pl.ds(i * TB, TB), :, w], so)

        def wait_out(i, trans_v, so):
            pltpu.make_async_copy(trans_v, out_hbm.at[pl.ds(i * TB, TB), :, w],
                                  so).wait()

        start_idx(0, idx_a, si_a)
        pltpu.make_async_copy(x_hbm.at[0, w, pl.ds(0, TB)], idx_a, si_a).wait()
        fire_gathers(idx_a, rows_a, sg_a)
        start_idx(1, idx_b, si_b)

        @pl.loop(0, NSTEP // 2)
        def _k(kk):
            i = 2 * kk
            pltpu.make_async_copy(
                x_hbm.at[(i + 1) // 2, w, pl.ds(((i + 1) % 2) * TB, TB)],
                idx_b, si_b).wait()
            fire_gathers(idx_b, rows_b, sg_b)
            wait_gathers(idx_a, rows_a, sg_a)
            @pl.when(kk < NSTEP // 2 - 1)
            def _():
                start_idx(i + 2, idx_a, si_a)
            @pl.when(kk > 0)
            def _():
                wait_out(i - 2, trans_a, so_a)
            compute(i, rows_a, trans_a)
            start_out(i, trans_a, so_a)

            j = i + 1
            @pl.when(kk < NSTEP // 2 - 1)
            def _():
                pltpu.make_async_copy(
                    x_hbm.at[(j + 1) // 2, w, pl.ds(((j + 1) % 2) * TB, TB)],
                    idx_a, si_a).wait()
                fire_gathers(idx_a, rows_a, sg_a)
            wait_gathers(idx_b, rows_b, sg_b)
            @pl.when(kk < NSTEP // 2 - 1)
            def _():
                start_idx(j + 2, idx_b, si_b)
            @pl.when(kk > 0)
            def _():
                wait_out(j - 2, trans_b, so_b)
            compute(j, rows_b, trans_b)
            start_out(j, trans_b, so_b)

        wait_out(NSTEP - 2, trans_a, so_a)
        wait_out(NSTEP - 1, trans_b, so_b)

    return k(x4, token_table, pos_table)


def kernel(x, token_table, pos_table):
    x4 = x.astype(jnp.int32).T.reshape(T // 8, 8, BT, BS).transpose(0, 2, 1, 3)
    out5 = _sc_embed(x4, token_table, pos_table)
    return out5.transpose(2, 4, 0, 1, 3).reshape(B, T, D)


# speedup vs baseline: 1.4509x; 1.1650x over previous
import functools

import jax
import jax.numpy as jnp
from jax import lax
from jax.experimental import pallas as pl
from jax.experimental.pallas import tpu as pltpu
from jax.experimental.pallas import tpu_sc as plsc

NC, NS = 2, 16
NW = NC * NS
B, T, D = 4096, 200, 32
TB = 4
NSTEP = T // TB
DT, DS = D // 8, 8
BT, BS = B // 128, 128


def _sc_embed(x4, token_table, pos_table):
    mesh = plsc.VectorSubcoreMesh(core_axis_name="c", subcore_axis_name="s")

    @functools.partial(
        pl.kernel,
        out_type=jax.ShapeDtypeStruct((T * DT, BT, DS * BS), jnp.float32),
        mesh=mesh,
        scratch_types=[
            pltpu.VMEM((TB, BS), jnp.int32),
            pltpu.VMEM((TB, BS), jnp.int32),
            pltpu.VMEM((TB * BS, D), jnp.float32),
            pltpu.VMEM((TB * BS, D), jnp.float32),
            pltpu.VMEM((TB * DT, DS * BS), jnp.float32),
            pltpu.VMEM((TB * DT, DS * BS), jnp.float32),
            pltpu.VMEM((T, D), jnp.float32),
            pltpu.SemaphoreType.DMA,
            pltpu.SemaphoreType.DMA,
            pltpu.SemaphoreType.DMA,
            pltpu.SemaphoreType.DMA,
            pltpu.SemaphoreType.DMA,
            pltpu.SemaphoreType.DMA,
        ],
        compiler_params=pltpu.CompilerParams(use_tc_tiling_on_sc=False,
                                             needs_layout_passes=False),
    )
    def k(x_hbm, tok_hbm, pos_hbm, out_hbm,
          idx_a, idx_b, rows_a, rows_b, trans_a, trans_b, pos_v,
          si_a, si_b, sg_a, sg_b, so_a, so_b):
        w = lax.axis_index("s") * NC + lax.axis_index("c")
        pltpu.sync_copy(pos_hbm, pos_v)

        iota = lax.iota(jnp.int32, 16)
        rowadd = [iota // 8 + h * 2 for h in (0, 1)]
        colbase = (iota % 8) * 128

        def start_idx(i, idx_v, si):
            tt = i // 2
            hs = (i % 2) * TB
            pltpu.async_copy(x_hbm.at[tt, w, pl.ds(hs, TB)], idx_v, si)

        def fire_gathers(idx_v, rows_v, sg):
            for r in range(TB):
                pltpu.async_copy(tok_hbm.at[idx_v.at[r]],
                                 rows_v.at[pl.ds(r * BS, BS)], sg)

        def wait_gathers(idx_v, rows_v, sg):
            for r in range(TB):
                pltpu.make_async_copy(tok_hbm.at[idx_v.at[r]],
                                      rows_v.at[pl.ds(r * BS, BS)], sg).wait()

        def compute(i, rows_v, trans_v):
            @pl.loop(0, TB)
            def _t(tm):
                t = i * TB + tm
                pv0 = pos_v[t, pl.ds(0, 16)]
                pv1 = pos_v[t, pl.ds(16, 16)]
                row0 = rowadd[0] + tm * TB
                row1 = rowadd[1] + tm * TB

                @pl.loop(0, BS, unroll=8)
                def _b(b_off):
                    r = tm * BS + b_off
                    col = colbase + b_off
                    v0 = rows_v[r, pl.ds(0, 16)] + pv0
                    v1 = rows_v[r, pl.ds(16, 16)] + pv1
                    plsc.store_scatter(trans_v, [row0, col], v0)
                    plsc.store_scatter(trans_v, [row1, col], v1)

        def start_out(i, trans_v, so):
            pltpu.async_copy(trans_v, out_hbm.at[pl.ds(i * TB * DT, TB * DT), w],
                             so)

        def wait_out(i, trans_v, so):
            pltpu.make_async_copy(trans_v,
                                  out_hbm.at[pl.ds(i * TB * DT, TB * DT), w],
                                  so).wait()

        start_idx(0, idx_a, si_a)
        pltpu.make_async_copy(x_hbm.at[0, w, pl.ds(0, TB)], idx_a, si_a).wait()
        fire_gathers(idx_a, rows_a, sg_a)
        start_idx(1, idx_b, si_b)

        @pl.loop(0, NSTEP // 2)
        def _k(kk):
            i = 2 * kk
            pltpu.make_async_copy(
                x_hbm.at[(i + 1) // 2, w, pl.ds(((i + 1) % 2) * TB, TB)],
                idx_b, si_b).wait()
            fire_gathers(idx_b, rows_b, sg_b)
            wait_gathers(idx_a, rows_a, sg_a)
            @pl.when(kk < NSTEP // 2 - 1)
            def _():
                start_idx(i + 2, idx_a, si_a)
            @pl.when(kk > 0)
            def _():
                wait_out(i - 2, trans_a, so_a)
            compute(i, rows_a, trans_a)
            start_out(i, trans_a, so_a)

            j = i + 1
            @pl.when(kk < NSTEP // 2 - 1)
            def _():
                pltpu.make_async_copy(
                    x_hbm.at[(j + 1) // 2, w, pl.ds(((j + 1) % 2) * TB, TB)],
                    idx_a, si_a).wait()
                fire_gathers(idx_a, rows_a, sg_a)
            wait_gathers(idx_b, rows_b, sg_b)
            @pl.when(kk < NSTEP // 2 - 1)
            def _():
                start_idx(j + 2, idx_b, si_b)
            @pl.when(kk > 0)
            def _():
                wait_out(j - 2, trans_b, so_b)
            compute(j, rows_b, trans_b)
            start_out(j, trans_b, so_b)

        wait_out(NSTEP - 2, trans_a, so_a)
        wait_out(NSTEP - 1, trans_b, so_b)

    return k(x4, token_table, pos_table)


def kernel(x, token_table, pos_table):
    x4 = x.astype(jnp.int32).T.reshape(T // 8, 8, BT, BS).transpose(0, 2, 1, 3)
    out3 = _sc_embed(x4, token_table, pos_table)
    out5 = out3.reshape(T, DT, BT, DS, BS)
    return out5.transpose(2, 4, 0, 1, 3).reshape(B, T, D)


# speedup vs baseline: 1.4866x; 1.0246x over previous
import functools

import jax
import jax.numpy as jnp
from jax import lax
from jax.experimental import pallas as pl
from jax.experimental.pallas import tpu as pltpu
from jax.experimental.pallas import tpu_sc as plsc

NC, NS = 2, 16
NW = NC * NS
B, T, D = 4096, 200, 32
TB = 4
NSTEP = T // TB
DT, DS = D // 8, 8
BT, BS = B // 128, 128


def _sc_embed(x4, token_table, pos_table):
    mesh = plsc.VectorSubcoreMesh(core_axis_name="c", subcore_axis_name="s")

    @functools.partial(
        pl.kernel,
        out_type=jax.ShapeDtypeStruct((T * DT, BT, DS * BS), jnp.float32),
        mesh=mesh,
        scratch_types=[
            pltpu.VMEM((TB, BS), jnp.int32),
            pltpu.VMEM((TB, BS), jnp.int32),
            pltpu.VMEM((TB * BS, D), jnp.float32),
            pltpu.VMEM((TB * BS, D), jnp.float32),
            pltpu.VMEM((TB * DT, DS * BS), jnp.float32),
            pltpu.VMEM((TB * DT, DS * BS), jnp.float32),
            pltpu.VMEM((TB * BS * 17,), jnp.float32),
            pltpu.VMEM((TB * BS * 17,), jnp.float32),
            pltpu.VMEM((T, D), jnp.float32),
            pltpu.SemaphoreType.DMA,
            pltpu.SemaphoreType.DMA,
            pltpu.SemaphoreType.DMA,
            pltpu.SemaphoreType.DMA,
            pltpu.SemaphoreType.DMA,
            pltpu.SemaphoreType.DMA,
        ],
        compiler_params=pltpu.CompilerParams(use_tc_tiling_on_sc=False,
                                             needs_layout_passes=False),
    )
    def k(x_hbm, tok_hbm, pos_hbm, out_hbm,
          idx_a, idx_b, rows_a, rows_b, trans_a, trans_b, p17lo, p17hi, pos_v,
          si_a, si_b, sg_a, sg_b, so_a, so_b):
        w = lax.axis_index("s") * NC + lax.axis_index("c")
        pltpu.sync_copy(pos_hbm, pos_v)

        iota = lax.iota(jnp.int32, 16)
        bidx = [(iota + bsg * 16) * 17 for bsg in range(8)]

        def start_idx(i, idx_v, si):
            tt = i // 2
            hs = (i % 2) * TB
            pltpu.async_copy(x_hbm.at[tt, w, pl.ds(hs, TB)], idx_v, si)

        def fire_gathers(idx_v, rows_v, sg):
            for r in range(TB):
                pltpu.async_copy(tok_hbm.at[idx_v.at[r]],
                                 rows_v.at[pl.ds(r * BS, BS)], sg)

        def wait_gathers(idx_v, rows_v, sg):
            for r in range(TB):
                pltpu.make_async_copy(tok_hbm.at[idx_v.at[r]],
                                      rows_v.at[pl.ds(r * BS, BS)], sg).wait()

        def compute(i, rows_v, trans_v):
            @pl.loop(0, TB)
            def _t(tm):
                t = i * TB + tm
                @pl.loop(0, BS, unroll=8)
                def _b(b_off):
                    r = tm * BS + b_off
                    p17lo[pl.ds(r * 17, 16)] = rows_v[r, pl.ds(0, 16)]
                    p17hi[pl.ds(r * 17, 16)] = rows_v[r, pl.ds(16, 16)]

                tsplat = jnp.full((16,), t, jnp.int32)
                for dh, buf in ((0, p17lo), (1, p17hi)):
                    for dl in range(16):
                        d = dh * 16 + dl
                        dsplat = jnp.full((16,), d, jnp.int32)
                        pvec = plsc.load_gather(pos_v, [tsplat, dsplat])
                        base = tm * BS * 17 + dl
                        for bsg in range(8):
                            v = plsc.load_gather(buf, [bidx[bsg] + base])
                            trans_v[tm * DT + d // 8,
                                    pl.ds((d % 8) * BS + bsg * 16, 16)] = v + pvec

        def start_out(i, trans_v, so):
            pltpu.async_copy(trans_v, out_hbm.at[pl.ds(i * TB * DT, TB * DT), w],
                             so)

        def wait_out(i, trans_v, so):
            pltpu.make_async_copy(trans_v,
                                  out_hbm.at[pl.ds(i * TB * DT, TB * DT), w],
                                  so).wait()

        start_idx(0, idx_a, si_a)
        pltpu.make_async_copy(x_hbm.at[0, w, pl.ds(0, TB)], idx_a, si_a).wait()
        fire_gathers(idx_a, rows_a, sg_a)
        start_idx(1, idx_b, si_b)

        @pl.loop(0, NSTEP // 2)
        def _k(kk):
            i = 2 * kk
            pltpu.make_async_copy(
                x_hbm.at[(i + 1) // 2, w, pl.ds(((i + 1) % 2) * TB, TB)],
                idx_b, si_b).wait()
            fire_gathers(idx_b, rows_b, sg_b)
            wait_gathers(idx_a, rows_a, sg_a)
            @pl.when(kk < NSTEP // 2 - 1)
            def _():
                start_idx(i + 2, idx_a, si_a)
            @pl.when(kk > 0)
            def _():
                wait_out(i - 2, trans_a, so_a)
            compute(i, rows_a, trans_a)
            start_out(i, trans_a, so_a)

            j = i + 1
            @pl.when(kk < NSTEP // 2 - 1)
            def _():
                pltpu.make_async_copy(
                    x_hbm.at[(j + 1) // 2, w, pl.ds(((j + 1) % 2) * TB, TB)],
                    idx_a, si_a).wait()
                fire_gathers(idx_a, rows_a, sg_a)
            wait_gathers(idx_b, rows_b, sg_b)
            @pl.when(kk < NSTEP // 2 - 1)
            def _():
                start_idx(j + 2, idx_b, si_b)
            @pl.when(kk > 0)
            def _():
                wait_out(j - 2, trans_b, so_b)
            compute(j, rows_b, trans_b)
            start_out(j, trans_b, so_b)

        wait_out(NSTEP - 2, trans_a, so_a)
        wait_out(NSTEP - 1, trans_b, so_b)

    return k(x4, token_table, pos_table)


def kernel(x, token_table, pos_table):
    x4 = x.astype(jnp.int32).T.reshape(T // 8, 8, BT, BS).transpose(0, 2, 1, 3)
    out3 = _sc_embed(x4, token_table, pos_table)
    out5 = out3.reshape(T, DT, BT, DS, BS)
    return out5.transpose(2, 4, 0, 1, 3).reshape(B, T, D)


# speedup vs baseline: 2.5116x; 1.6895x over previous
import functools

import jax
import jax.numpy as jnp
from jax import lax
from jax.experimental import pallas as pl
from jax.experimental.pallas import tpu as pltpu
from jax.experimental.pallas import tpu_sc as plsc

NC, NS = 2, 16
NW = NC * NS
B, T, D = 4096, 200, 32
TB = 4
NSTEP = T // TB
DT, DS = D // 8, 8
BT, BS = B // 128, 128


def _sc_embed(x4, token_table, pos_table):
    mesh = plsc.VectorSubcoreMesh(core_axis_name="c", subcore_axis_name="s")

    @functools.partial(
        pl.kernel,
        out_type=jax.ShapeDtypeStruct((T * DT, BT, DS * BS), jnp.float32),
        mesh=mesh,
        scratch_types=[
            pltpu.VMEM((TB, BS), jnp.int32),
            pltpu.VMEM((TB, BS), jnp.int32),
            pltpu.VMEM((TB * BS, D), jnp.float32),
            pltpu.VMEM((TB * BS, D), jnp.float32),
            pltpu.VMEM((TB * DT, DS * BS), jnp.float32),
            pltpu.VMEM((TB * DT, DS * BS), jnp.float32),
            pltpu.VMEM((2, TB * BS * 17), jnp.float32),
            pltpu.VMEM((T, D), jnp.float32),
            pltpu.SemaphoreType.DMA,
            pltpu.SemaphoreType.DMA,
            pltpu.SemaphoreType.DMA,
            pltpu.SemaphoreType.DMA,
            pltpu.SemaphoreType.DMA,
            pltpu.SemaphoreType.DMA,
        ],
        compiler_params=pltpu.CompilerParams(use_tc_tiling_on_sc=False,
                                             needs_layout_passes=False,
                                             disable_bounds_checks=True),
    )
    def k(x_hbm, tok_hbm, pos_hbm, out_hbm,
          idx_a, idx_b, rows_a, rows_b, trans_a, trans_b, p17, pos_v,
          si_a, si_b, sg_a, sg_b, so_a, so_b):
        w = lax.axis_index("s") * NC + lax.axis_index("c")
        pltpu.sync_copy(pos_hbm, pos_v)

        iota = lax.iota(jnp.int32, 16)
        bidx = [(iota + bsg * 16) * 17 for bsg in range(8)]

        def start_idx(i, idx_v, si):
            tt = i // 2
            hs = (i % 2) * TB
            pltpu.async_copy(x_hbm.at[tt, w, pl.ds(hs, TB)], idx_v, si)

        def fire_gathers(idx_v, rows_v, sg):
            for r in range(TB):
                pltpu.async_copy(tok_hbm.at[idx_v.at[r]],
                                 rows_v.at[pl.ds(r * BS, BS)], sg)

        def wait_gathers(idx_v, rows_v, sg):
            for r in range(TB):
                pltpu.make_async_copy(tok_hbm.at[idx_v.at[r]],
                                      rows_v.at[pl.ds(r * BS, BS)], sg).wait()

        def compute(i, rows_v, trans_v):
            @pl.loop(0, TB)
            def _t(tm):
                t = i * TB + tm

                @plsc.parallel_loop(0, BS, unroll=8)
                def _b(b_off):
                    r = tm * BS + b_off
                    p17[0, pl.ds(r * 17, 16)] = rows_v[r, pl.ds(0, 16)]
                    p17[1, pl.ds(r * 17, 16)] = rows_v[r, pl.ds(16, 16)]

                tsplat = jnp.full((16,), t, jnp.int32)

                @plsc.parallel_loop(0, D, unroll=4)
                def _d(d):
                    dh = d // 16
                    dl = d - dh * 16
                    dsplat = jnp.full((16,), 0, jnp.int32) + d
                    pvec = plsc.load_gather(pos_v, [tsplat, dsplat])
                    buf = p17.at[dh]
                    base = tm * BS * 17 + dl
                    trow = tm * DT + d // 8
                    tcol = (d % 8) * BS
                    for bsg in range(8):
                        v = plsc.load_gather(buf, [bidx[bsg] + base])
                        trans_v[trow, pl.ds(tcol + bsg * 16, 16)] = v + pvec

        def start_out(i, trans_v, so):
            pltpu.async_copy(trans_v, out_hbm.at[pl.ds(i * TB * DT, TB * DT), w],
                             so)

        def wait_out(i, trans_v, so):
            pltpu.make_async_copy(trans_v,
                                  out_hbm.at[pl.ds(i * TB * DT, TB * DT), w],
                                  so).wait()

        start_idx(0, idx_a, si_a)
        pltpu.make_async_copy(x_hbm.at[0, w, pl.ds(0, TB)], idx_a, si_a).wait()
        fire_gathers(idx_a, rows_a, sg_a)
        start_idx(1, idx_b, si_b)

        @pl.loop(0, NSTEP // 2)
        def _k(kk):
            i = 2 * kk
            pltpu.make_async_copy(
                x_hbm.at[(i + 1) // 2, w, pl.ds(((i + 1) % 2) * TB, TB)],
                idx_b, si_b).wait()
            fire_gathers(idx_b, rows_b, sg_b)
            wait_gathers(idx_a, rows_a, sg_a)
            @pl.when(kk < NSTEP // 2 - 1)
            def _():
                start_idx(i + 2, idx_a, si_a)
            @pl.when(kk > 0)
            def _():
                wait_out(i - 2, trans_a, so_a)
            compute(i, rows_a, trans_a)
            start_out(i, trans_a, so_a)

            j = i + 1
            @pl.when(kk < NSTEP // 2 - 1)
            def _():
                pltpu.make_async_copy(
                    x_hbm.at[(j + 1) // 2, w, pl.ds(((j + 1) % 2) * TB, TB)],
                    idx_a, si_a).wait()
                fire_gathers(idx_a, rows_a, sg_a)
            wait_gathers(idx_b, rows_b, sg_b)
            @pl.when(kk < NSTEP // 2 - 1)
            def _():
                start_idx(j + 2, idx_b, si_b)
            @pl.when(kk > 0)
            def _():
                wait_out(j - 2, trans_b, so_b)
            compute(j, rows_b, trans_b)
            start_out(j, trans_b, so_b)

        wait_out(NSTEP - 2, trans_a, so_a)
        wait_out(NSTEP - 1, trans_b, so_b)

    return k(x4, token_table, pos_table)


def kernel(x, token_table, pos_table):
    x4 = x.astype(jnp.int32).T.reshape(T // 8, 8, BT, BS).transpose(0, 2, 1, 3)
    out3 = _sc_embed(x4, token_table, pos_table)
    out5 = out3.reshape(T, DT, BT, DS, BS)
    return out5.transpose(2, 4, 0, 1, 3).reshape(B, T, D)


# speedup vs baseline: 2.5150x; 1.0013x over previous
import functools

import jax
import jax.numpy as jnp
from jax import lax
from jax.experimental import pallas as pl
from jax.experimental.pallas import tpu as pltpu
from jax.experimental.pallas import tpu_sc as plsc

NC, NS = 2, 16
NW = NC * NS
B, T, D = 4096, 200, 32
TB = 4
NSTEP = T // TB
DT, DS = D // 8, 8
BT, BS = B // 128, 128


def _sc_embed(x4, token_table, pos_table):
    mesh = plsc.VectorSubcoreMesh(core_axis_name="c", subcore_axis_name="s")

    @functools.partial(
        pl.kernel,
        out_type=jax.ShapeDtypeStruct((T * DT, BT, DS * BS), jnp.float32),
        mesh=mesh,
        scratch_types=[
            pltpu.VMEM((TB, BS), jnp.int32),
            pltpu.VMEM((TB, BS), jnp.int32),
            pltpu.VMEM((TB * BS, D), jnp.float32),
            pltpu.VMEM((TB * BS, D), jnp.float32),
            pltpu.VMEM((TB * DT, DS * BS), jnp.float32),
            pltpu.VMEM((TB * DT, DS * BS), jnp.float32),
            pltpu.VMEM((2, TB * BS * 17), jnp.float32),
            pltpu.VMEM((T, D), jnp.float32),
            pltpu.SemaphoreType.DMA,
            pltpu.SemaphoreType.DMA,
            pltpu.SemaphoreType.DMA,
            pltpu.SemaphoreType.DMA,
            pltpu.SemaphoreType.DMA,
            pltpu.SemaphoreType.DMA,
            pltpu.SemaphoreType.REGULAR,
        ],
        compiler_params=pltpu.CompilerParams(use_tc_tiling_on_sc=False,
                                             needs_layout_passes=False,
                                             disable_bounds_checks=True),
    )
    def k(x_hbm, tok_hbm, pos_hbm, out_hbm,
          idx_a, idx_b, rows_a, rows_b, trans_a, trans_b, p17, pos_v,
          si_a, si_b, sg_a, sg_b, so_a, so_b, fence_sem):
        w = lax.axis_index("s") * NC + lax.axis_index("c")
        pltpu.sync_copy(pos_hbm, pos_v)

        iota = lax.iota(jnp.int32, 16)
        bidx = [(iota + bsg * 16) * 17 for bsg in range(8)]

        def start_idx(i, idx_v, si):
            tt = i // 2
            hs = (i % 2) * TB
            pltpu.async_copy(x_hbm.at[tt, w, pl.ds(hs, TB)], idx_v, si)

        def fire_gathers(idx_v, rows_v, sg):
            for r in range(TB):
                pltpu.async_copy(tok_hbm.at[idx_v.at[r]],
                                 rows_v.at[pl.ds(r * BS, BS)], sg)

        def wait_gathers(idx_v, rows_v, sg):
            for r in range(TB):
                pltpu.make_async_copy(tok_hbm.at[idx_v.at[r]],
                                      rows_v.at[pl.ds(r * BS, BS)], sg).wait()

        def compute(i, rows_v, trans_v):
            @pl.loop(0, TB)
            def _t(tm):
                t = i * TB + tm

                @plsc.parallel_loop(0, BS, unroll=8)
                def _b(b_off):
                    r = tm * BS + b_off
                    p17[0, pl.ds(r * 17, 16)] = rows_v[r, pl.ds(0, 16)]
                    p17[1, pl.ds(r * 17, 16)] = rows_v[r, pl.ds(16, 16)]

                tsplat = jnp.full((16,), t, jnp.int32)

                @plsc.parallel_loop(0, D, unroll=4)
                def _d(d):
                    dh = d // 16
                    dl = d - dh * 16
                    dsplat = jnp.full((16,), 0, jnp.int32) + d
                    pvec = plsc.load_gather(pos_v, [tsplat, dsplat])
                    buf = p17.at[dh]
                    base = tm * BS * 17 + dl
                    trow = tm * DT + d // 8
                    tcol = (d % 8) * BS
                    for bsg in range(8):
                        v = plsc.load_gather(buf, [bidx[bsg] + base])
                        trans_v[trow, pl.ds(tcol + bsg * 16, 16)] = v + pvec

        def start_out(i, trans_v, so):
            pltpu.semaphore_signal(fence_sem, 1)
            pl.semaphore_wait(fence_sem, 1)
            pltpu.async_copy(trans_v, out_hbm.at[pl.ds(i * TB * DT, TB * DT), w],
                             so)

        def wait_out(i, trans_v, so):
            pltpu.make_async_copy(trans_v,
                                  out_hbm.at[pl.ds(i * TB * DT, TB * DT), w],
                                  so).wait()

        start_idx(0, idx_a, si_a)
        pltpu.make_async_copy(x_hbm.at[0, w, pl.ds(0, TB)], idx_a, si_a).wait()
        fire_gathers(idx_a, rows_a, sg_a)
        start_idx(1, idx_b, si_b)

        @pl.loop(0, NSTEP // 2)
        def _k(kk):
            i = 2 * kk
            pltpu.make_async_copy(
                x_hbm.at[(i + 1) // 2, w, pl.ds(((i + 1) % 2) * TB, TB)],
                idx_b, si_b).wait()
            fire_gathers(idx_b, rows_b, sg_b)
            wait_gathers(idx_a, rows_a, sg_a)
            @pl.when(kk < NSTEP // 2 - 1)
            def _():
                start_idx(i + 2, idx_a, si_a)
            @pl.when(kk > 0)
            def _():
                wait_out(i - 2, trans_a, so_a)
            compute(i, rows_a, trans_a)
            start_out(i, trans_a, so_a)

            j = i + 1
            @pl.when(kk < NSTEP // 2 - 1)
            def _():
                pltpu.make_async_copy(
                    x_hbm.at[(j + 1) // 2, w, pl.ds(((j + 1) % 2) * TB, TB)],
                    idx_a, si_a).wait()
                fire_gathers(idx_a, rows_a, sg_a)
            wait_gathers(idx_b, rows_b, sg_b)
            @pl.when(kk < NSTEP // 2 - 1)
            def _():
                start_idx(j + 2, idx_b, si_b)
            @pl.when(kk > 0)
            def _():
                wait_out(j - 2, trans_b, so_b)
            compute(j, rows_b, trans_b)
            start_out(j, trans_b, so_b)

        wait_out(NSTEP - 2, trans_a, so_a)
        wait_out(NSTEP - 1, trans_b, so_b)

    return k(x4, token_table, pos_table)


def kernel(x, token_table, pos_table):
    x4 = x.astype(jnp.int32).T.reshape(T // 8, 8, BT, BS).transpose(0, 2, 1, 3)
    out3 = _sc_embed(x4, token_table, pos_table)
    out5 = out3.reshape(T, DT, BT, DS, BS)
    return out5.transpose(2, 4, 0, 1, 3).reshape(B, T, D)


# speedup vs baseline: 2.5177x; 1.0011x over previous
import functools

import jax
import jax.numpy as jnp
from jax import lax
from jax.experimental import pallas as pl
from jax.experimental.pallas import tpu as pltpu
from jax.experimental.pallas import tpu_sc as plsc

NC, NS = 2, 16
NW = NC * NS
B, T, D = 4096, 200, 32
TB = 4
NSTEP = T // TB
DT, DS = D // 8, 8
BT, BS = B // 128, 128

def _sc_embed(x4, token_table, pos_table):
    mesh = plsc.VectorSubcoreMesh(core_axis_name="c", subcore_axis_name="s")

    @functools.partial(
        pl.kernel,
        out_type=jax.ShapeDtypeStruct((T * DT, BT, DS * BS), jnp.float32),
        mesh=mesh,
        scratch_types=[
            pltpu.VMEM((TB, BS), jnp.int32),
            pltpu.VMEM((TB, BS), jnp.int32),
            pltpu.VMEM((TB * BS, D), jnp.float32),
            pltpu.VMEM((TB * BS, D), jnp.float32),
            pltpu.VMEM((TB * DT, DS * BS), jnp.float32),
            pltpu.VMEM((TB * DT, DS * BS), jnp.float32),
            pltpu.VMEM((D * 513,), jnp.float32),
            pltpu.VMEM((T, D), jnp.float32),
            pltpu.SemaphoreType.DMA,
            pltpu.SemaphoreType.DMA,
            pltpu.SemaphoreType.DMA,
            pltpu.SemaphoreType.DMA,
            pltpu.SemaphoreType.DMA,
            pltpu.SemaphoreType.DMA,
            pltpu.SemaphoreType.REGULAR,
        ],
        compiler_params=pltpu.CompilerParams(use_tc_tiling_on_sc=False,
                                             needs_layout_passes=False,
                                             disable_bounds_checks=True),
    )
    def k(x_hbm, tok_hbm, pos_hbm, out_hbm,
          idx_a, idx_b, rows_a, rows_b, trans_a, trans_b, stg, pos_v,
          si_a, si_b, sg_a, sg_b, so_a, so_b, fence_sem):
        w = lax.axis_index("s") * NC + lax.axis_index("c")
        pltpu.sync_copy(pos_hbm, pos_v)

        iota = lax.iota(jnp.int32, 16)
        ia = [(iota + h * 16) * 513 for h in (0, 1)]

        def start_idx(i, idx_v, si):
            tt = i // 2
            hs = (i % 2) * TB
            pltpu.async_copy(x_hbm.at[tt, w, pl.ds(hs, TB)], idx_v, si)

        def fire_gathers(idx_v, rows_v, sg):
            for r in range(TB):
                pltpu.async_copy(tok_hbm.at[idx_v.at[r]],
                                 rows_v.at[pl.ds(r * BS, BS)], sg)

        def wait_gathers(idx_v, rows_v, sg):
            for r in range(TB):
                pltpu.make_async_copy(tok_hbm.at[idx_v.at[r]],
                                      rows_v.at[pl.ds(r * BS, BS)], sg).wait()

        def fence():
            pltpu.semaphore_signal(fence_sem, 1)
            pl.semaphore_wait(fence_sem, 1)

        def compute(i, rows_v, trans_v):
            @pl.loop(0, TB)
            def _t(tm):
                t = i * TB + tm
                pv0 = pos_v[t, pl.ds(0, 16)]
                pv1 = pos_v[t, pl.ds(16, 16)]

                @plsc.parallel_loop(0, BS, unroll=8)
                def _b(b_off):
                    r = tm * BS + b_off
                    plsc.store_scatter(stg, [ia[0] + r],
                                       rows_v[r, pl.ds(0, 16)] + pv0)
                    plsc.store_scatter(stg, [ia[1] + r],
                                       rows_v[r, pl.ds(16, 16)] + pv1)

                fence()

                @plsc.parallel_loop(0, D, unroll=4)
                def _d(d):
                    base = d * 513 + tm * BS
                    trow = tm * DT + d // 8
                    tcol = (d % 8) * BS
                    for bsg in range(8):
                        trans_v[trow, pl.ds(tcol + bsg * 16, 16)] = (
                            stg[pl.ds(base + bsg * 16, 16)])

        def start_out(i, trans_v, so):
            fence()
            pltpu.async_copy(trans_v, out_hbm.at[pl.ds(i * TB * DT, TB * DT), w],
                             so)

        def wait_out(i, trans_v, so):
            pltpu.make_async_copy(trans_v,
                                  out_hbm.at[pl.ds(i * TB * DT, TB * DT), w],
                                  so).wait()

        start_idx(0, idx_a, si_a)
        pltpu.make_async_copy(x_hbm.at[0, w, pl.ds(0, TB)], idx_a, si_a).wait()
        fire_gathers(idx_a, rows_a, sg_a)
        start_idx(1, idx_b, si_b)

        @pl.loop(0, NSTEP // 2)
        def _k(kk):
            i = 2 * kk
            pltpu.make_async_copy(
                x_hbm.at[(i + 1) // 2, w, pl.ds(((i + 1) % 2) * TB, TB)],
                idx_b, si_b).wait()
            fire_gathers(idx_b, rows_b, sg_b)
            wait_gathers(idx_a, rows_a, sg_a)
            @pl.when(kk < NSTEP // 2 - 1)
            def _():
                start_idx(i + 2, idx_a, si_a)
            @pl.when(kk > 0)
            def _():
                wait_out(i - 2, trans_a, so_a)
            compute(i, rows_a, trans_a)
            start_out(i, trans_a, so_a)

            j = i + 1
            @pl.when(kk < NSTEP // 2 - 1)
            def _():
                pltpu.make_async_copy(
                    x_hbm.at[(j + 1) // 2, w, pl.ds(((j + 1) % 2) * TB, TB)],
                    idx_a, si_a).wait()
                fire_gathers(idx_a, rows_a, sg_a)
            wait_gathers(idx_b, rows_b, sg_b)
            @pl.when(kk < NSTEP // 2 - 1)
            def _():
                start_idx(j + 2, idx_b, si_b)
            @pl.when(kk > 0)
            def _():
                wait_out(j - 2, trans_b, so_b)
            compute(j, rows_b, trans_b)
            start_out(j, trans_b, so_b)

        wait_out(NSTEP - 2, trans_a, so_a)
        wait_out(NSTEP - 1, trans_b, so_b)

    return k(x4, token_table, pos_table)


def kernel(x, token_table, pos_table):
    x4 = x.astype(jnp.int32).T.reshape(T // 8, 8, BT, BS).transpose(0, 2, 1, 3)
    out3 = _sc_embed(x4, token_table, pos_table)
    out5 = out3.reshape(T, DT, BT, DS, BS)
    return out5.transpose(2, 4, 0, 1, 3).reshape(B, T, D)
